# baseline (speedup 1.0000x reference)
import functools

import jax
import jax.numpy as jnp
from jax import lax
from jax.experimental import pallas as pl
from jax.experimental.pallas import tpu as pltpu
from jax.experimental.pallas import tpu_sc as plsc

_S, _T, _D = 16384, 200, 128
_B = _S * _T

_info = plsc.get_sparse_core_info()
_NC, _NS = _info.num_cores, _info.num_subcores
_NW = _NC * _NS
_BW = _B // _NW
_CHUNK = 128
_NCHUNK = _BW // _CHUNK


def kernel(emotion_indices, table):
    idx_flat = emotion_indices.reshape(_B).astype(jnp.int32)
    mesh = plsc.VectorSubcoreMesh(core_axis_name="c", subcore_axis_name="s")

    @functools.partial(
        pl.kernel,
        mesh=mesh,
        out_type=jax.ShapeDtypeStruct((_B, _D), jnp.float32),
        scratch_types=[
            pltpu.VMEM((_CHUNK,), jnp.int32),
            pltpu.VMEM((_CHUNK, _D), jnp.float32),
            pltpu.SemaphoreType.DMA,
        ],
    )
    def sc_gather(table_hbm, idx_hbm, out_hbm, idx_v, rows_v, sem):
        wid = lax.axis_index("s") * _NC + lax.axis_index("c")
        base = wid * _BW

        def body(i, carry):
            off = base + i * _CHUNK
            pltpu.sync_copy(idx_hbm.at[pl.ds(off, _CHUNK)], idx_v)
            pltpu.async_copy(table_hbm.at[idx_v], rows_v, sem).wait()
            pltpu.sync_copy(rows_v, out_hbm.at[pl.ds(off, _CHUNK)])
            return carry

        lax.fori_loop(0, _NCHUNK, body, 0)

    out = sc_gather(table, idx_flat)
    return out.reshape(_S, _T, _D)


# speedup vs baseline: 1.0065x; 1.0065x over previous
import functools

import jax
import jax.numpy as jnp
from jax import lax
from jax.experimental import pallas as pl
from jax.experimental.pallas import tpu as pltpu
from jax.experimental.pallas import tpu_sc as plsc

_S, _T, _D = 16384, 200, 128
_B = _S * _T

_info = plsc.get_sparse_core_info()
_NC, _NS = _info.num_cores, _info.num_subcores
_NW = _NC * _NS
_BW = _B // _NW
_CHUNK = 128
_NBUF = 5
_NCHUNK = _BW // _CHUNK
_NROUND = _NCHUNK // _NBUF


def kernel(emotion_indices, table):
    idx_flat = emotion_indices.reshape(_B).astype(jnp.int32)
    mesh = plsc.VectorSubcoreMesh(core_axis_name="c", subcore_axis_name="s")

    @functools.partial(
        pl.kernel,
        mesh=mesh,
        out_type=jax.ShapeDtypeStruct((_B, _D), jnp.float32),
        scratch_types=[
            pltpu.VMEM((_NBUF * _CHUNK,), jnp.int32),
            pltpu.VMEM((_NBUF * _CHUNK,), jnp.int32),
            pltpu.VMEM((_NBUF, _CHUNK, _D), jnp.float32),
            pltpu.SemaphoreType.DMA,
            pltpu.SemaphoreType.DMA,
        ]
        + [pltpu.SemaphoreType.DMA] * _NBUF
        + [pltpu.SemaphoreType.DMA] * _NBUF,
    )
    def sc_gather(table_hbm, idx_hbm, out_hbm, idx0, idx1, rows,
                  sem_i0, sem_i1, *sems):
        sem_g = sems[:_NBUF]
        sem_o = sems[_NBUF:]
        idx_bufs = (idx0, idx1)
        sem_i = (sem_i0, sem_i1)

        wid = lax.axis_index("s") * _NC + lax.axis_index("c")
        chunk0 = wid * _NCHUNK
        rnd_idx = _NBUF * _CHUNK

        def start_idx_load(r, p):
            pltpu.async_copy(
                idx_hbm.at[pl.ds((chunk0 + r * _NBUF) * _CHUNK, rnd_idx)],
                idx_bufs[p], sem_i[p])

        def wait_idx(p):
            pltpu.make_async_copy(idx_hbm.at[pl.ds(chunk0 * _CHUNK, rnd_idx)],
                                  idx_bufs[p], sem_i[p]).wait()

        def start_gather(p, b):
            pltpu.async_copy(table_hbm.at[idx_bufs[p].at[pl.ds(b * _CHUNK, _CHUNK)]],
                             rows.at[b], sem_g[b])

        def wait_gather(p, b):
            pltpu.make_async_copy(table_hbm.at[idx_bufs[p].at[pl.ds(b * _CHUNK, _CHUNK)]],
                                  rows.at[b], sem_g[b]).wait()

        def out_slice(r, b):
            return out_hbm.at[pl.ds((chunk0 + r * _NBUF + b) * _CHUNK, _CHUNK)]

        def start_out(r, b):
            pltpu.async_copy(rows.at[b], out_slice(r, b), sem_o[b])

        def wait_out(r, b):
            pltpu.make_async_copy(rows.at[b], out_slice(r, b), sem_o[b]).wait()

        def round_body(r, p, first):
            start_idx_load(r + 1, 1 - p)
            if not first:
                wait_idx(p)
            for b in range(_NBUF):
                if not first:
                    wait_out(r, b)
                start_gather(p, b)
            for b in range(_NBUF):
                wait_gather(p, b)
                start_out(r, b)

        pltpu.sync_copy(idx_hbm.at[pl.ds(chunk0 * _CHUNK, rnd_idx)], idx_bufs[0])
        round_body(0, 0, first=True)

        def body(k, carry):
            r = 1 + 2 * k
            round_body(r, 1, first=False)
            round_body(r + 1, 0, first=False)
            return carry

        lax.fori_loop(0, (_NROUND - 2) // 2, body, 0)

        r_last = _NROUND - 1
        wait_idx(1)
        for b in range(_NBUF):
            wait_out(r_last, b)
            start_gather(1, b)
        for b in range(_NBUF):
            wait_gather(1, b)
            start_out(r_last, b)
        for b in range(_NBUF):
            wait_out(r_last, b)

    out = sc_gather(table, idx_flat)
    return out.reshape(_S, _T, _D)


# speedup vs baseline: 5.2509x; 5.2171x over previous
import functools

import jax
import jax.numpy as jnp
from jax import lax
from jax.experimental import pallas as pl
from jax.experimental.pallas import tpu as pltpu
from jax.experimental.pallas import tpu_sc as plsc

_S, _T, _D = 16384, 200, 128
_B = _S * _T

_info = plsc.get_sparse_core_info()
_NC, _NS = _info.num_cores, _info.num_subcores
_NW = _NC * _NS
_BW = _B // _NW
_CHUNK = 128
_NBUF = 5
_NCHUNK = _BW // _CHUNK
_NROUND = _NCHUNK // _NBUF


def kernel(emotion_indices, table):
    idx_flat = emotion_indices.reshape(_B).astype(jnp.int32)
    table_rep = jnp.tile(table, (_NW, 1))
    mesh = plsc.VectorSubcoreMesh(core_axis_name="c", subcore_axis_name="s")

    @functools.partial(
        pl.kernel,
        mesh=mesh,
        out_type=jax.ShapeDtypeStruct((_B, _D), jnp.float32),
        scratch_types=[
            pltpu.VMEM((_NBUF * _CHUNK,), jnp.int32),
            pltpu.VMEM((_NBUF * _CHUNK,), jnp.int32),
            pltpu.VMEM((_NBUF, _CHUNK, _D), jnp.float32),
            pltpu.SemaphoreType.DMA,
            pltpu.SemaphoreType.DMA,
        ]
        + [pltpu.SemaphoreType.DMA] * _NBUF
        + [pltpu.SemaphoreType.DMA] * _NBUF,
    )
    def sc_gather(table_hbm, idx_hbm, out_hbm, idx0, idx1, rows,
                  sem_i0, sem_i1, *sems):
        sem_g = sems[:_NBUF]
        sem_o = sems[_NBUF:]
        idx_bufs = (idx0, idx1)
        sem_i = (sem_i0, sem_i1)

        wid = lax.axis_index("s") * _NC + lax.axis_index("c")
        chunk0 = wid * _NCHUNK
        rnd_idx = _NBUF * _CHUNK

        def start_idx_load(r, p):
            pltpu.async_copy(
                idx_hbm.at[pl.ds((chunk0 + r * _NBUF) * _CHUNK, rnd_idx)],
                idx_bufs[p], sem_i[p])

        def wait_idx(p):
            pltpu.make_async_copy(idx_hbm.at[pl.ds(chunk0 * _CHUNK, rnd_idx)],
                                  idx_bufs[p], sem_i[p]).wait()

        def start_gather(p, b):
            pltpu.async_copy(table_hbm.at[idx_bufs[p].at[pl.ds(b * _CHUNK, _CHUNK)]],
                             rows.at[b], sem_g[b])

        def wait_gather(p, b):
            pltpu.make_async_copy(table_hbm.at[idx_bufs[p].at[pl.ds(b * _CHUNK, _CHUNK)]],
                                  rows.at[b], sem_g[b]).wait()

        def out_slice(r, b):
            return out_hbm.at[pl.ds((chunk0 + r * _NBUF + b) * _CHUNK, _CHUNK)]

        def start_out(r, b):
            pltpu.async_copy(rows.at[b], out_slice(r, b), sem_o[b])

        def wait_out(r, b):
            pltpu.make_async_copy(rows.at[b], out_slice(r, b), sem_o[b]).wait()

        def bias_idx(p):
            off = jnp.full((16,), 9, jnp.int32) * wid
            for k in range(rnd_idx // 16):
                sl = pl.ds(k * 16, 16)
                idx_bufs[p][sl] = idx_bufs[p][sl] + off

        def round_body(r, p, first):
            start_idx_load(r + 1, 1 - p)
            if not first:
                wait_idx(p)
            bias_idx(p)
            for b in range(_NBUF):
                if not first:
                    wait_out(r, b)
                start_gather(p, b)
            for b in range(_NBUF):
                wait_gather(p, b)
                start_out(r, b)

        pltpu.sync_copy(idx_hbm.at[pl.ds(chunk0 * _CHUNK, rnd_idx)], idx_bufs[0])
        round_body(0, 0, first=True)

        def body(k, carry):
            r = 1 + 2 * k
            round_body(r, 1, first=False)
            round_body(r + 1, 0, first=False)
            return carry

        lax.fori_loop(0, (_NROUND - 2) // 2, body, 0)

        r_last = _NROUND - 1
        wait_idx(1)
        bias_idx(1)
        for b in range(_NBUF):
            wait_out(r_last, b)
            start_gather(1, b)
        for b in range(_NBUF):
            wait_gather(1, b)
            start_out(r_last, b)
        for b in range(_NBUF):
            wait_out(r_last, b)

    out = sc_gather(table_rep, idx_flat)
    return out.reshape(_S, _T, _D)


# speedup vs baseline: 13.3338x; 2.5394x over previous
import functools

import jax
import jax.numpy as jnp
from jax import lax
from jax.experimental import pallas as pl
from jax.experimental.pallas import tpu as pltpu
from jax.experimental.pallas import tpu_sc as plsc

_S, _T, _D = 16384, 200, 128
_B = _S * _T

_info = plsc.get_sparse_core_info()
_NC, _NS = _info.num_cores, _info.num_subcores
_NW = _NC * _NS
_BW = _B // _NW
_CHUNK = 128
_NBUF = 5
_NCHUNK = _BW // _CHUNK
_NROUND = _NCHUNK // _NBUF


def kernel(emotion_indices, table):
    idx_flat = emotion_indices.reshape(_B).astype(jnp.int32)
    table_rep = jnp.tile(table, (_NW * _NBUF * 16, 1))
    mesh = plsc.VectorSubcoreMesh(core_axis_name="c", subcore_axis_name="s")

    @functools.partial(
        pl.kernel,
        mesh=mesh,
        out_type=jax.ShapeDtypeStruct((_B, _D), jnp.float32),
        scratch_types=[
            pltpu.VMEM((_NBUF * _CHUNK,), jnp.int32),
            pltpu.VMEM((_NBUF * _CHUNK,), jnp.int32),
            pltpu.VMEM((_NBUF, _CHUNK, _D), jnp.float32),
            pltpu.SemaphoreType.DMA,
            pltpu.SemaphoreType.DMA,
        ]
        + [pltpu.SemaphoreType.DMA] * _NBUF
        + [pltpu.SemaphoreType.DMA] * _NBUF,
    )
    def sc_gather(table_hbm, idx_hbm, out_hbm, idx0, idx1, rows,
                  sem_i0, sem_i1, *sems):
        sem_g = sems[:_NBUF]
        sem_o = sems[_NBUF:]
        idx_bufs = (idx0, idx1)
        sem_i = (sem_i0, sem_i1)

        wid = lax.axis_index("s") * _NC + lax.axis_index("c")
        chunk0 = wid * _NCHUNK
        rnd_idx = _NBUF * _CHUNK

        def start_idx_load(r, p):
            pltpu.async_copy(
                idx_hbm.at[pl.ds((chunk0 + r * _NBUF) * _CHUNK, rnd_idx)],
                idx_bufs[p], sem_i[p])

        def wait_idx(p):
            pltpu.make_async_copy(idx_hbm.at[pl.ds(chunk0 * _CHUNK, rnd_idx)],
                                  idx_bufs[p], sem_i[p]).wait()

        def start_gather(p, b):
            pltpu.async_copy(table_hbm.at[idx_bufs[p].at[pl.ds(b * _CHUNK, _CHUNK)]],
                             rows.at[b], sem_g[b])

        def wait_gather(p, b):
            pltpu.make_async_copy(table_hbm.at[idx_bufs[p].at[pl.ds(b * _CHUNK, _CHUNK)]],
                                  rows.at[b], sem_g[b]).wait()

        def out_slice(r, b):
            return out_hbm.at[pl.ds((chunk0 + r * _NBUF + b) * _CHUNK, _CHUNK)]

        def start_out(r, b):
            pltpu.async_copy(rows.at[b], out_slice(r, b), sem_o[b])

        def wait_out(r, b):
            pltpu.make_async_copy(rows.at[b], out_slice(r, b), sem_o[b]).wait()

        def bias_idx(p):
            lane = lax.iota(jnp.int32, 16)
            base = jnp.full((16,), 9 * _NBUF * 16, jnp.int32) * wid
            for k in range(rnd_idx // 16):
                b = k // (_CHUNK // 16)
                off = base + (9 * 16 * b) + 9 * lane
                sl = pl.ds(k * 16, 16)
                idx_bufs[p][sl] = idx_bufs[p][sl] + off

        def round_body(r, p, first):
            start_idx_load(r + 1, 1 - p)
            if not first:
                wait_idx(p)
            bias_idx(p)
            for b in range(_NBUF):
                if not first:
                    wait_out(r, b)
                start_gather(p, b)
            for b in range(_NBUF):
                wait_gather(p, b)
                start_out(r, b)

        pltpu.sync_copy(idx_hbm.at[pl.ds(chunk0 * _CHUNK, rnd_idx)], idx_bufs[0])
        round_body(0, 0, first=True)

        def body(k, carry):
            r = 1 + 2 * k
            round_body(r, 1, first=False)
            round_body(r + 1, 0, first=False)
            return carry

        lax.fori_loop(0, (_NROUND - 2) // 2, body, 0)

        r_last = _NROUND - 1
        wait_idx(1)
        bias_idx(1)
        for b in range(_NBUF):
            wait_out(r_last, b)
            start_gather(1, b)
        for b in range(_NBUF):
            wait_gather(1, b)
            start_out(r_last, b)
        for b in range(_NBUF):
            wait_out(r_last, b)

    out = sc_gather(table_rep, idx_flat)
    return out.reshape(_S, _T, _D)
